# baseline (speedup 1.0000x reference)
import functools

import jax
import jax.numpy as jnp
from jax import lax
from jax.experimental import pallas as pl
from jax.experimental.pallas import tpu as pltpu
from jax.experimental.pallas import tpu_sc as plsc

_LEFT = 32
_TILE = 256
_WIN = 384
_CHUNK_S = 1024
_CHUNK_R = _CHUNK_S + _TILE


def _sc_gather(M, C, idx_pad, base):
    V, D = M.shape
    info = plsc.get_sparse_core_info()
    nc, ns = info.num_cores, info.num_subcores
    nw = nc * ns
    b_per_w = _CHUNK_R // nw
    ch = 8
    depth = 6
    lag = 3
    mesh = plsc.VectorSubcoreMesh(core_axis_name="c", subcore_axis_name="s")

    @functools.partial(
        pl.kernel,
        mesh=mesh,
        out_type=[
            jax.ShapeDtypeStruct((_CHUNK_R, D), jnp.float32),
            jax.ShapeDtypeStruct((_CHUNK_R, D), jnp.float32),
        ],
        scratch_types=[
            pltpu.VMEM((b_per_w,), jnp.int32),
            pltpu.VMEM((depth, ch, D), jnp.float32),
        ]
        + [pltpu.SemaphoreType.DMA] * (2 * depth),
    )
    def gather_kernel(m_hbm, c_hbm, idx_hbm, outm_hbm, outc_hbm,
                      idx_v, rows_v, *sems):
        g_sems = sems[:depth]
        w_sems = sems[depth:]
        c = lax.axis_index("c")
        s = lax.axis_index("s")
        wid = s * nc + c
        wbase = wid * b_per_w
        pltpu.sync_copy(idx_hbm.at[pl.ds(base + wbase, b_per_w)], idx_v)

        jobs = []
        for u in range(b_per_w // ch):
            jobs.append((m_hbm, outm_hbm, u * ch))
            jobs.append((c_hbm, outc_hbm, u * ch))
        g_cp = [None] * len(jobs)
        w_cp = [None] * len(jobs)

        def writeback(j):
            src, dst, off = jobs[j]
            slot = j % depth
            g_cp[j].wait()
            w_cp[j] = pltpu.async_copy(
                rows_v.at[slot], dst.at[pl.ds(wbase + off, ch)], w_sems[slot])

        for j, (src, dst, off) in enumerate(jobs):
            slot = j % depth
            if j >= depth:
                w_cp[j - depth].wait()
            g_cp[j] = pltpu.async_copy(
                src.at[idx_v.at[pl.ds(off, ch)]], rows_v.at[slot], g_sems[slot])
            if j >= lag:
                writeback(j - lag)
        for j in range(max(len(jobs) - lag, 0), len(jobs)):
            writeback(j)
        for j in range(max(len(jobs) - depth, 0), len(jobs)):
            w_cp[j].wait()

    return gather_kernel(M, C, idx_pad)


def _attn_body(*refs):
    if len(refs) == 11:
        e_ref, m0_ref, m1_ref, m2_ref, c0_ref, c1_ref, c2_ref, out_ref, p_ref \
            = refs[2:]
    else:
        e_ref, m0_ref, m1_ref, m2_ref, c0_ref, c1_ref, c2_ref, out_ref, p_ref \
            = refs
    E = e_ref[0]
    Wm = jnp.concatenate([m0_ref[...], m1_ref[...], m2_ref[...]], axis=0)
    A = lax.dot_general(E, Wm, (((1,), (1,)), ((), ())),
                        preferred_element_type=jnp.float32)
    t_i = lax.broadcasted_iota(jnp.int32, (_TILE, _WIN), 0)
    c_i = lax.broadcasted_iota(jnp.int32, (_TILE, _WIN), 1)
    delta = c_i - t_i
    band = (delta >= 0) & (delta < _LEFT)
    Am = jnp.where(band, A, -1e30)
    m = jnp.max(Am, axis=1, keepdims=True)
    ex = jnp.exp(Am - m)
    denom = jnp.sum(ex, axis=1, keepdims=True)
    pn = ex / denom
    Wc = jnp.concatenate([c0_ref[...], c1_ref[...], c2_ref[...]], axis=0)
    comp = lax.dot_general(pn, Wc, (((1,), (0,)), ((), ())),
                           preferred_element_type=jnp.float32)
    D = E.shape[1]
    out_ref[0, :, :D] = comp
    out_ref[0, :, D:] = E
    x = pn
    for b in range(8):
        k = 1 << b
        rolled = jnp.concatenate([x[:, k:], x[:, :k]], axis=1)
        bit = (lax.broadcasted_iota(jnp.int32, (_TILE, _WIN), 0) >> b) & 1
        x = jnp.where(bit == 1, rolled, x)
    p_ref[0] = x[:, :_LEFT].T


def _attn_tc(enc, Mrows, Crows, out_prev, p_prev, tile_off):
    B, S, D = enc.shape
    ntiles = _CHUNK_S // _TILE
    data_specs = [
        pl.BlockSpec((1, _TILE, D), lambda i: (0, i + tile_off, 0)),
        pl.BlockSpec((128, D), lambda i: (2 * i, 0)),
        pl.BlockSpec((128, D), lambda i: (2 * i + 1, 0)),
        pl.BlockSpec((128, D), lambda i: (2 * i + 2, 0)),
        pl.BlockSpec((128, D), lambda i: (2 * i, 0)),
        pl.BlockSpec((128, D), lambda i: (2 * i + 1, 0)),
        pl.BlockSpec((128, D), lambda i: (2 * i + 2, 0)),
    ]
    data_args = (enc, Mrows, Mrows, Mrows, Crows, Crows, Crows)
    if out_prev is None:
        in_specs, args, aliases = data_specs, data_args, {}
    else:
        in_specs = [pl.BlockSpec(memory_space=pl.ANY),
                    pl.BlockSpec(memory_space=pl.ANY)] + data_specs
        args = (out_prev, p_prev) + data_args
        aliases = {0: 0, 1: 1}
    out, p = pl.pallas_call(
        _attn_body,
        grid=(ntiles,),
        in_specs=in_specs,
        out_specs=[
            pl.BlockSpec((1, _TILE, 2 * D), lambda i: (0, i + tile_off, 0)),
            pl.BlockSpec((1, _LEFT, _TILE), lambda i: (0, 0, i + tile_off)),
        ],
        out_shape=[
            jax.ShapeDtypeStruct((1, S, 2 * D), jnp.float32),
            jax.ShapeDtypeStruct((1, _LEFT, S), jnp.float32),
        ],
        input_output_aliases=aliases,
    )(*args)
    return out, p


def kernel(symbols, encodings, M, C):
    B, S = symbols.shape
    D = encodings.shape[-1]
    sym = symbols[0].astype(jnp.int32)
    n_pad = ((S + _LEFT - 1) // _TILE + 1) * _TILE
    idx_pad = lax.pad(sym, sym[0], [(_LEFT - 1, n_pad - S - (_LEFT - 1), 0)])
    Mr0, Cr0 = _sc_gather(M, C, idx_pad, 0)
    Mr1, Cr1 = _sc_gather(M, C, idx_pad, _CHUNK_S)
    out0 = jnp.zeros((1, S, 2 * D), jnp.float32)
    p0 = jnp.zeros((1, _LEFT, S), jnp.float32)
    out0, p0 = _attn_tc(encodings, Mr0, Cr0, out0, p0, 0)
    out, p = _attn_tc(encodings, Mr1, Cr1, out0, p0, _CHUNK_S // _TILE)
    return out, jnp.transpose(p, (0, 2, 1))
